# initial kernel scaffold (unmeasured)
import jax
import jax.numpy as jnp
from jax import lax
from jax.experimental import pallas as pl
from jax.experimental.pallas import tpu as pltpu

N_DEV = 4


def kernel(x, w_mat):
    m_per, k = x.shape
    k2, n_per = w_mat.shape

    def body(x_ref, w_ref, out_ref, comm_ref, send_sems, recv_sems):
        my_pos = lax.axis_index("i")
        left = (my_pos - 1) % N_DEV
        right = (my_pos + 1) % N_DEV

        barrier_sem = pltpu.get_barrier_semaphore()
        for nbr in [left, right]:
            pl.semaphore_signal(
                barrier_sem, inc=1,
                device_id=(nbr,), device_id_type=pl.DeviceIdType.MESH,
            )
        pl.semaphore_wait(barrier_sem, 2)

        comm_ref[0, :, :] = x_ref[:, :]
        out_ref[pl.ds(my_pos * m_per, m_per), :] = jnp.dot(
            x_ref[:, :], w_ref[:, :], preferred_element_type=jnp.float32
        )

        for h in range(N_DEV - 1):
            send_slot = h % 2
            recv_slot = (h + 1) % 2
            rdma = pltpu.make_async_remote_copy(
                src_ref=comm_ref.at[send_slot],
                dst_ref=comm_ref.at[recv_slot],
                send_sem=send_sems.at[send_slot],
                recv_sem=recv_sems.at[recv_slot],
                device_id=(right,),
                device_id_type=pl.DeviceIdType.MESH,
            )
            rdma.start()
            rdma.wait()

            origin = (my_pos - h - 1) % N_DEV
            out_ref[pl.ds(origin * m_per, m_per), :] = jnp.dot(
                comm_ref[recv_slot, :, :], w_ref[:, :],
                preferred_element_type=jnp.float32,
            )

    return pl.pallas_call(
        body,
        out_shape=jax.ShapeDtypeStruct((N_DEV * m_per, n_per), jnp.float32),
        in_specs=[
            pl.BlockSpec(memory_space=pltpu.VMEM),
            pl.BlockSpec(memory_space=pltpu.VMEM),
        ],
        out_specs=pl.BlockSpec(memory_space=pltpu.VMEM),
        scratch_shapes=[
            pltpu.VMEM((2, m_per, k), x.dtype),
            pltpu.SemaphoreType.DMA((2,)),
            pltpu.SemaphoreType.DMA((2,)),
        ],
        compiler_params=pltpu.CompilerParams(collective_id=0),
    )(x, w_mat)


# baseline (device time: 327190 ns/iter reference)
import jax
import jax.numpy as jnp
from jax import lax
from jax.experimental import pallas as pl
from jax.experimental.pallas import tpu as pltpu

N_DEV = 4


def kernel(x, w_mat):
    x = x.astype(jnp.bfloat16)
    w_mat = w_mat.astype(jnp.bfloat16)
    m_per, k = x.shape
    k2, n_per = w_mat.shape

    def body(x_ref, w_ref, out_ref, comm_ref, send_sems, recv_sems):
        my_pos = lax.axis_index("i")
        left = (my_pos - 1) % N_DEV
        right = (my_pos + 1) % N_DEV

        barrier_sem = pltpu.get_barrier_semaphore()
        for nbr in [left, right]:
            pl.semaphore_signal(
                barrier_sem, inc=1,
                device_id=(nbr,), device_id_type=pl.DeviceIdType.MESH,
            )
        pl.semaphore_wait(barrier_sem, 2)

        comm_ref[0, :, :] = x_ref[:, :]
        out_ref[pl.ds(my_pos * m_per, m_per), :] = jnp.dot(
            x_ref[:, :], w_ref[:, :], preferred_element_type=jnp.float32
        )

        for h in range(N_DEV - 1):
            send_slot = h % 2
            recv_slot = (h + 1) % 2
            rdma = pltpu.make_async_remote_copy(
                src_ref=comm_ref.at[send_slot],
                dst_ref=comm_ref.at[recv_slot],
                send_sem=send_sems.at[send_slot],
                recv_sem=recv_sems.at[recv_slot],
                device_id=(right,),
                device_id_type=pl.DeviceIdType.MESH,
            )
            rdma.start()
            rdma.wait()

            origin = (my_pos - h - 1) % N_DEV
            out_ref[pl.ds(origin * m_per, m_per), :] = jnp.dot(
                comm_ref[recv_slot, :, :], w_ref[:, :],
                preferred_element_type=jnp.float32,
            )

    return pl.pallas_call(
        body,
        out_shape=jax.ShapeDtypeStruct((N_DEV * m_per, n_per), jnp.float32),
        in_specs=[
            pl.BlockSpec(memory_space=pltpu.VMEM),
            pl.BlockSpec(memory_space=pltpu.VMEM),
        ],
        out_specs=pl.BlockSpec(memory_space=pltpu.VMEM),
        scratch_shapes=[
            pltpu.VMEM((2, m_per, k), x.dtype),
            pltpu.SemaphoreType.DMA((2,)),
            pltpu.SemaphoreType.DMA((2,)),
        ],
        compiler_params=pltpu.CompilerParams(
            collective_id=0, vmem_limit_bytes=100 * 1024 * 1024,
        ),
    )(x, w_mat)


# device time: 175405 ns/iter; 1.8653x vs baseline; 1.8653x over previous
import jax
import jax.numpy as jnp
from jax import lax
from jax.experimental import pallas as pl
from jax.experimental.pallas import tpu as pltpu

N_DEV = 4


def kernel(x, w_mat):
    x = x.astype(jnp.bfloat16)
    w_mat = w_mat.astype(jnp.bfloat16)
    m_per, k = x.shape
    k2, n_per = w_mat.shape
    m_half = m_per // 2

    def body(x_ref, w_ref, out_ref, bufL, bufR, bufO,
             send_sems, recv_sems):
        my_pos = lax.axis_index("i")
        left = (my_pos - 1) % N_DEV
        right = (my_pos + 1) % N_DEV

        barrier_sem = pltpu.get_barrier_semaphore()
        for nbr in [left, right]:
            pl.semaphore_signal(
                barrier_sem, inc=1,
                device_id=(nbr,), device_id_type=pl.DeviceIdType.MESH,
            )
        pl.semaphore_wait(barrier_sem, 2)

        s1r = pltpu.make_async_remote_copy(
            src_ref=x_ref, dst_ref=bufL,
            send_sem=send_sems.at[0], recv_sem=recv_sems.at[0],
            device_id=(right,), device_id_type=pl.DeviceIdType.MESH,
        )
        s1l = pltpu.make_async_remote_copy(
            src_ref=x_ref, dst_ref=bufR,
            send_sem=send_sems.at[1], recv_sem=recv_sems.at[1],
            device_id=(left,), device_id_type=pl.DeviceIdType.MESH,
        )
        s1r.start()
        s1l.start()

        out_ref[pl.ds(my_pos * m_per, m_per), :] = jnp.dot(
            x_ref[:, :], w_ref[:, :], preferred_element_type=jnp.float32
        )

        s1r.wait_recv()
        s2r = pltpu.make_async_remote_copy(
            src_ref=bufL.at[pl.ds(0, m_half)],
            dst_ref=bufO.at[pl.ds(0, m_half)],
            send_sem=send_sems.at[2], recv_sem=recv_sems.at[2],
            device_id=(right,), device_id_type=pl.DeviceIdType.MESH,
        )
        s2r.start()

        s1l.wait_recv()
        s2l = pltpu.make_async_remote_copy(
            src_ref=bufR.at[pl.ds(m_half, m_half)],
            dst_ref=bufO.at[pl.ds(m_half, m_half)],
            send_sem=send_sems.at[3], recv_sem=recv_sems.at[3],
            device_id=(left,), device_id_type=pl.DeviceIdType.MESH,
        )
        s2l.start()

        out_ref[pl.ds(left * m_per, m_per), :] = jnp.dot(
            bufL[:, :], w_ref[:, :], preferred_element_type=jnp.float32
        )
        out_ref[pl.ds(right * m_per, m_per), :] = jnp.dot(
            bufR[:, :], w_ref[:, :], preferred_element_type=jnp.float32
        )

        opp = (my_pos + 2) % N_DEV
        s2r.wait_recv()
        out_ref[pl.ds(opp * m_per, m_half), :] = jnp.dot(
            bufO[pl.ds(0, m_half), :], w_ref[:, :],
            preferred_element_type=jnp.float32,
        )
        s2l.wait_recv()
        out_ref[pl.ds(opp * m_per + m_half, m_half), :] = jnp.dot(
            bufO[pl.ds(m_half, m_half), :], w_ref[:, :],
            preferred_element_type=jnp.float32,
        )

        s1r.wait_send()
        s1l.wait_send()
        s2r.wait_send()
        s2l.wait_send()

    return pl.pallas_call(
        body,
        out_shape=jax.ShapeDtypeStruct((N_DEV * m_per, n_per), jnp.float32),
        in_specs=[
            pl.BlockSpec(memory_space=pltpu.VMEM),
            pl.BlockSpec(memory_space=pltpu.VMEM),
        ],
        out_specs=pl.BlockSpec(memory_space=pltpu.VMEM),
        scratch_shapes=[
            pltpu.VMEM((m_per, k), jnp.bfloat16),
            pltpu.VMEM((m_per, k), jnp.bfloat16),
            pltpu.VMEM((m_per, k), jnp.bfloat16),
            pltpu.SemaphoreType.DMA((4,)),
            pltpu.SemaphoreType.DMA((4,)),
        ],
        compiler_params=pltpu.CompilerParams(
            collective_id=0, vmem_limit_bytes=100 * 1024 * 1024,
        ),
    )(x, w_mat)


# device time: 166490 ns/iter; 1.9652x vs baseline; 1.0535x over previous
import jax
import jax.numpy as jnp
from jax import lax
from jax.experimental import pallas as pl
from jax.experimental.pallas import tpu as pltpu

N_DEV = 4


def kernel(x, w_mat):
    w_mat = w_mat.astype(jnp.bfloat16)
    m_per, k = x.shape
    k2, n_per = w_mat.shape
    m_half = m_per // 2
    m_q = m_per // 4

    def body(x_ref, w_ref, out_ref, xbf, bufL, bufR, bufO,
             send_sems, recv_sems):
        my_pos = lax.axis_index("i")
        left = (my_pos - 1) % N_DEV
        right = (my_pos + 1) % N_DEV

        xbf[:, :] = x_ref[:, :].astype(jnp.bfloat16)

        barrier_sem = pltpu.get_barrier_semaphore()
        for nbr in [left, right]:
            pl.semaphore_signal(
                barrier_sem, inc=1,
                device_id=(nbr,), device_id_type=pl.DeviceIdType.MESH,
            )
        pl.semaphore_wait(barrier_sem, 2)

        s1r = pltpu.make_async_remote_copy(
            src_ref=xbf, dst_ref=bufL,
            send_sem=send_sems.at[0], recv_sem=recv_sems.at[0],
            device_id=(right,), device_id_type=pl.DeviceIdType.MESH,
        )
        s1l = pltpu.make_async_remote_copy(
            src_ref=xbf, dst_ref=bufR,
            send_sem=send_sems.at[1], recv_sem=recv_sems.at[1],
            device_id=(left,), device_id_type=pl.DeviceIdType.MESH,
        )
        s1r.start()
        s1l.start()

        out_ref[pl.ds(my_pos * m_per, m_per), :] = jnp.dot(
            xbf[:, :], w_ref[:, :], preferred_element_type=jnp.float32
        )

        s1r.wait_recv()
        fwd_r = []
        for q in range(2):
            f = pltpu.make_async_remote_copy(
                src_ref=bufL.at[pl.ds(q * m_q, m_q)],
                dst_ref=bufO.at[pl.ds(q * m_q, m_q)],
                send_sem=send_sems.at[2 + q], recv_sem=recv_sems.at[2 + q],
                device_id=(right,), device_id_type=pl.DeviceIdType.MESH,
            )
            f.start()
            fwd_r.append(f)

        s1l.wait_recv()
        fwd_l = []
        for q in range(2):
            f = pltpu.make_async_remote_copy(
                src_ref=bufR.at[pl.ds(m_half + q * m_q, m_q)],
                dst_ref=bufO.at[pl.ds(m_half + q * m_q, m_q)],
                send_sem=send_sems.at[4 + q], recv_sem=recv_sems.at[4 + q],
                device_id=(left,), device_id_type=pl.DeviceIdType.MESH,
            )
            f.start()
            fwd_l.append(f)

        out_ref[pl.ds(left * m_per, m_per), :] = jnp.dot(
            bufL[:, :], w_ref[:, :], preferred_element_type=jnp.float32
        )
        out_ref[pl.ds(right * m_per, m_per), :] = jnp.dot(
            bufR[:, :], w_ref[:, :], preferred_element_type=jnp.float32
        )

        opp = (my_pos + 2) % N_DEV
        for q, f in [(0, fwd_r[0]), (2, fwd_l[0]), (1, fwd_r[1]), (3, fwd_l[1])]:
            f.wait_recv()
            out_ref[pl.ds(opp * m_per + q * m_q, m_q), :] = jnp.dot(
                bufO[pl.ds(q * m_q, m_q), :], w_ref[:, :],
                preferred_element_type=jnp.float32,
            )

        for f in [s1r, s1l] + fwd_r + fwd_l:
            f.wait_send()

    return pl.pallas_call(
        body,
        out_shape=jax.ShapeDtypeStruct((N_DEV * m_per, n_per), jnp.float32),
        in_specs=[
            pl.BlockSpec(memory_space=pltpu.VMEM),
            pl.BlockSpec(memory_space=pltpu.VMEM),
        ],
        out_specs=pl.BlockSpec(memory_space=pltpu.VMEM),
        scratch_shapes=[
            pltpu.VMEM((m_per, k), jnp.bfloat16),
            pltpu.VMEM((m_per, k), jnp.bfloat16),
            pltpu.VMEM((m_per, k), jnp.bfloat16),
            pltpu.VMEM((m_per, k), jnp.bfloat16),
            pltpu.SemaphoreType.DMA((6,)),
            pltpu.SemaphoreType.DMA((6,)),
        ],
        compiler_params=pltpu.CompilerParams(
            collective_id=0, vmem_limit_bytes=100 * 1024 * 1024,
        ),
    )(x, w_mat)


# device time: 120222 ns/iter; 2.7215x vs baseline; 1.3849x over previous
import jax
import jax.numpy as jnp
from jax import lax
from jax.experimental import pallas as pl
from jax.experimental.pallas import tpu as pltpu

N_DEV = 4


def kernel(x, w_mat):
    w_mat = w_mat.astype(jnp.bfloat16)
    m_per, k = x.shape
    k2, n_per = w_mat.shape
    n_half = n_per // 2

    def body(x_ref, w_ref, out_ref, xbf,
             wL, wR, wOa, wOb,
             bBl, bBr, bBoa, bBob,
             fwdA, fwdB,
             bInL, bInR, bOa, bOb,
             send_sems, recv_sems):
        my_pos = lax.axis_index("i")
        left = (my_pos - 1) % N_DEV
        right = (my_pos + 1) % N_DEV
        opp = (my_pos + 2) % N_DEV

        def rdma(src, dst, i, dev):
            return pltpu.make_async_remote_copy(
                src_ref=src, dst_ref=dst,
                send_sem=send_sems.at[i], recv_sem=recv_sems.at[i],
                device_id=(dev,), device_id_type=pl.DeviceIdType.MESH,
            )

        barrier_sem = pltpu.get_barrier_semaphore()
        for nbr in [left, right]:
            pl.semaphore_signal(
                barrier_sem, inc=1,
                device_id=(nbr,), device_id_type=pl.DeviceIdType.MESH,
            )
        pl.semaphore_wait(barrier_sem, 2)

        sW_r = rdma(w_ref, wL, 0, right)
        sW_l = rdma(w_ref, wR, 1, left)
        sW_r.start()
        sW_l.start()

        xbf[:, :] = x_ref[:, :].astype(jnp.bfloat16)
        out_ref[pl.ds(my_pos * m_per, m_per), :] = jnp.dot(
            xbf[:, :], w_ref[:, :], preferred_element_type=jnp.float32
        )

        sW_r.wait_recv()
        fW_r = rdma(wL.at[:, pl.ds(0, n_half)], wOa, 2, right)
        fW_r.start()
        sW_l.wait_recv()
        fW_l = rdma(wR.at[:, pl.ds(n_half, n_half)], wOb, 3, left)
        fW_l.start()

        bBl[:, :] = jnp.dot(
            xbf[:, :], wL[:, :], preferred_element_type=jnp.float32
        ).astype(jnp.bfloat16)
        sB_l = rdma(bBl, bInR, 4, left)
        sB_l.start()
        bBr[:, :] = jnp.dot(
            xbf[:, :], wR[:, :], preferred_element_type=jnp.float32
        ).astype(jnp.bfloat16)
        sB_r = rdma(bBr, bInL, 5, right)
        sB_r.start()

        fW_r.wait_recv()
        bBoa[:, :] = jnp.dot(
            xbf[:, :], wOa[:, :], preferred_element_type=jnp.float32
        ).astype(jnp.bfloat16)
        sBo_a = rdma(bBoa, fwdA, 6, right)
        sBo_a.start()
        fW_l.wait_recv()
        bBob[:, :] = jnp.dot(
            xbf[:, :], wOb[:, :], preferred_element_type=jnp.float32
        ).astype(jnp.bfloat16)
        sBo_b = rdma(bBob, fwdB, 7, left)
        sBo_b.start()

        sB_l.wait_recv()
        out_ref[pl.ds(right * m_per, m_per), :] = bInR[:, :].astype(jnp.float32)
        sB_r.wait_recv()
        out_ref[pl.ds(left * m_per, m_per), :] = bInL[:, :].astype(jnp.float32)

        sBo_a.wait_recv()
        fB_r = rdma(fwdA, bOa, 8, right)
        fB_r.start()
        sBo_b.wait_recv()
        fB_l = rdma(fwdB, bOb, 9, left)
        fB_l.start()

        fB_r.wait_recv()
        out_ref[pl.ds(opp * m_per, m_per), pl.ds(0, n_half)] = (
            bOa[:, :].astype(jnp.float32)
        )
        fB_l.wait_recv()
        out_ref[pl.ds(opp * m_per, m_per), pl.ds(n_half, n_half)] = (
            bOb[:, :].astype(jnp.float32)
        )

        for s in [sW_r, sW_l, fW_r, fW_l, sB_l, sB_r,
                  sBo_a, sBo_b, fB_r, fB_l]:
            s.wait_send()

    bf = jnp.bfloat16
    return pl.pallas_call(
        body,
        out_shape=jax.ShapeDtypeStruct((N_DEV * m_per, n_per), jnp.float32),
        in_specs=[
            pl.BlockSpec(memory_space=pltpu.VMEM),
            pl.BlockSpec(memory_space=pltpu.VMEM),
        ],
        out_specs=pl.BlockSpec(memory_space=pltpu.VMEM),
        scratch_shapes=[
            pltpu.VMEM((m_per, k), bf),
            pltpu.VMEM((k, n_per), bf),
            pltpu.VMEM((k, n_per), bf),
            pltpu.VMEM((k, n_half), bf),
            pltpu.VMEM((k, n_half), bf),
            pltpu.VMEM((m_per, n_per), bf),
            pltpu.VMEM((m_per, n_per), bf),
            pltpu.VMEM((m_per, n_half), bf),
            pltpu.VMEM((m_per, n_half), bf),
            pltpu.VMEM((m_per, n_half), bf),
            pltpu.VMEM((m_per, n_half), bf),
            pltpu.VMEM((m_per, n_per), bf),
            pltpu.VMEM((m_per, n_per), bf),
            pltpu.VMEM((m_per, n_half), bf),
            pltpu.VMEM((m_per, n_half), bf),
            pltpu.SemaphoreType.DMA((10,)),
            pltpu.SemaphoreType.DMA((10,)),
        ],
        compiler_params=pltpu.CompilerParams(
            collective_id=0, vmem_limit_bytes=100 * 1024 * 1024,
        ),
    )(x, w_mat)


# device time: 112427 ns/iter; 2.9102x vs baseline; 1.0693x over previous
import jax
import jax.numpy as jnp
from jax import lax
from jax.experimental import pallas as pl
from jax.experimental.pallas import tpu as pltpu

N_DEV = 4


def kernel(x, w_mat):
    w_mat = w_mat.astype(jnp.bfloat16)
    m_per, k = x.shape
    k2, n_per = w_mat.shape
    n_half = n_per // 2

    def body(x_hbm, w_ref, out_hbm, xf32, xbf,
             wL, wR, wOa, wOb,
             bBl, bBr, bBoa, bBob,
             fwdA, fwdB,
             bInL, bInR, bOa, bOb,
             st_me, st_l, st_r, st_oa, st_ob,
             send_sems, recv_sems, local_sems):
        my_pos = lax.axis_index("i")
        left = (my_pos - 1) % N_DEV
        right = (my_pos + 1) % N_DEV
        opp = (my_pos + 2) % N_DEV

        def rdma(src, dst, i, dev):
            return pltpu.make_async_remote_copy(
                src_ref=src, dst_ref=dst,
                send_sem=send_sems.at[i], recv_sem=recv_sems.at[i],
                device_id=(dev,), device_id_type=pl.DeviceIdType.MESH,
            )

        x_load = pltpu.make_async_copy(x_hbm, xf32, local_sems.at[0])
        x_load.start()

        barrier_sem = pltpu.get_barrier_semaphore()
        for nbr in [left, right]:
            pl.semaphore_signal(
                barrier_sem, inc=1,
                device_id=(nbr,), device_id_type=pl.DeviceIdType.MESH,
            )
        pl.semaphore_wait(barrier_sem, 2)

        sW_r = rdma(w_ref, wL, 0, right)
        sW_l = rdma(w_ref, wR, 1, left)
        sW_r.start()
        sW_l.start()

        x_load.wait()
        xbf[:, :] = xf32[:, :].astype(jnp.bfloat16)
        st_me[:, :] = jnp.dot(
            xbf[:, :], w_ref[:, :], preferred_element_type=jnp.float32
        )
        c_me = pltpu.make_async_copy(
            st_me, out_hbm.at[pl.ds(my_pos * m_per, m_per), :],
            local_sems.at[1],
        )
        c_me.start()

        sW_r.wait_recv()
        fW_r = rdma(wL.at[:, pl.ds(0, n_half)], wOa, 2, right)
        fW_r.start()
        sW_l.wait_recv()
        fW_l = rdma(wR.at[:, pl.ds(n_half, n_half)], wOb, 3, left)
        fW_l.start()

        bBl[:, :] = jnp.dot(
            xbf[:, :], wL[:, :], preferred_element_type=jnp.float32
        ).astype(jnp.bfloat16)
        sB_l = rdma(bBl, bInR, 4, left)
        sB_l.start()
        bBr[:, :] = jnp.dot(
            xbf[:, :], wR[:, :], preferred_element_type=jnp.float32
        ).astype(jnp.bfloat16)
        sB_r = rdma(bBr, bInL, 5, right)
        sB_r.start()

        fW_r.wait_recv()
        bBoa[:, :] = jnp.dot(
            xbf[:, :], wOa[:, :], preferred_element_type=jnp.float32
        ).astype(jnp.bfloat16)
        sBo_a = rdma(bBoa, fwdA, 6, right)
        sBo_a.start()
        fW_l.wait_recv()
        bBob[:, :] = jnp.dot(
            xbf[:, :], wOb[:, :], preferred_element_type=jnp.float32
        ).astype(jnp.bfloat16)
        sBo_b = rdma(bBob, fwdB, 7, left)
        sBo_b.start()

        sB_l.wait_recv()
        st_r[:, :] = bInR[:, :].astype(jnp.float32)
        c_r = pltpu.make_async_copy(
            st_r, out_hbm.at[pl.ds(right * m_per, m_per), :],
            local_sems.at[2],
        )
        c_r.start()
        sB_r.wait_recv()
        st_l[:, :] = bInL[:, :].astype(jnp.float32)
        c_l = pltpu.make_async_copy(
            st_l, out_hbm.at[pl.ds(left * m_per, m_per), :],
            local_sems.at[3],
        )
        c_l.start()

        sBo_a.wait_recv()
        fB_r = rdma(fwdA, bOa, 8, right)
        fB_r.start()
        sBo_b.wait_recv()
        fB_l = rdma(fwdB, bOb, 9, left)
        fB_l.start()

        fB_r.wait_recv()
        st_oa[:, :] = bOa[:, :].astype(jnp.float32)
        c_oa = pltpu.make_async_copy(
            st_oa, out_hbm.at[pl.ds(opp * m_per, m_per), pl.ds(0, n_half)],
            local_sems.at[4],
        )
        c_oa.start()
        fB_l.wait_recv()
        st_ob[:, :] = bOb[:, :].astype(jnp.float32)
        c_ob = pltpu.make_async_copy(
            st_ob,
            out_hbm.at[pl.ds(opp * m_per, m_per), pl.ds(n_half, n_half)],
            local_sems.at[5],
        )
        c_ob.start()

        for c in [c_me, c_r, c_l, c_oa, c_ob]:
            c.wait()
        for s in [sW_r, sW_l, fW_r, fW_l, sB_l, sB_r,
                  sBo_a, sBo_b, fB_r, fB_l]:
            s.wait_send()

    bf = jnp.bfloat16
    f32 = jnp.float32
    return pl.pallas_call(
        body,
        out_shape=jax.ShapeDtypeStruct((N_DEV * m_per, n_per), f32),
        in_specs=[
            pl.BlockSpec(memory_space=pl.ANY),
            pl.BlockSpec(memory_space=pltpu.VMEM),
        ],
        out_specs=pl.BlockSpec(memory_space=pl.ANY),
        scratch_shapes=[
            pltpu.VMEM((m_per, k), f32),
            pltpu.VMEM((m_per, k), bf),
            pltpu.VMEM((k, n_per), bf),
            pltpu.VMEM((k, n_per), bf),
            pltpu.VMEM((k, n_half), bf),
            pltpu.VMEM((k, n_half), bf),
            pltpu.VMEM((m_per, n_per), bf),
            pltpu.VMEM((m_per, n_per), bf),
            pltpu.VMEM((m_per, n_half), bf),
            pltpu.VMEM((m_per, n_half), bf),
            pltpu.VMEM((m_per, n_half), bf),
            pltpu.VMEM((m_per, n_half), bf),
            pltpu.VMEM((m_per, n_per), bf),
            pltpu.VMEM((m_per, n_per), bf),
            pltpu.VMEM((m_per, n_half), bf),
            pltpu.VMEM((m_per, n_half), bf),
            pltpu.VMEM((m_per, n_per), f32),
            pltpu.VMEM((m_per, n_per), f32),
            pltpu.VMEM((m_per, n_per), f32),
            pltpu.VMEM((m_per, n_half), f32),
            pltpu.VMEM((m_per, n_half), f32),
            pltpu.SemaphoreType.DMA((10,)),
            pltpu.SemaphoreType.DMA((10,)),
            pltpu.SemaphoreType.DMA((6,)),
        ],
        compiler_params=pltpu.CompilerParams(
            collective_id=0, vmem_limit_bytes=100 * 1024 * 1024,
        ),
    )(x, w_mat)
